# baseline (speedup 1.0000x reference)
import functools

import jax
import jax.numpy as jnp
from jax import lax
from jax.experimental import pallas as pl
from jax.experimental.pallas import tpu as pltpu
from jax.experimental.pallas import tpu_sc as plsc

_K = 2
_NC = 2
_NW = 16 * _NC


@jax.jit
def _sc_gather(x, table):
    bsz, p = x.shape
    d = table.shape[1]
    b_per_w = bsz // _NW
    n_chunks = b_per_w
    n_rounds = n_chunks // _K
    mesh = plsc.VectorSubcoreMesh(
        core_axis_name="c", subcore_axis_name="s", num_cores=_NC
    )

    @functools.partial(
        pl.kernel,
        out_type=jax.ShapeDtypeStruct((bsz, p, d), jnp.float32),
        mesh=mesh,
        scratch_types=[
            pltpu.VMEM((b_per_w, p), jnp.int32),
            pltpu.VMEM((2, _K, p, d), jnp.float32),
            pltpu.SemaphoreType.DMA,
            pltpu.SemaphoreType.DMA,
        ],
        compiler_params=pltpu.CompilerParams(
            use_tc_tiling_on_sc=False, skip_device_barrier=True
        ),
    )
    def gather_kernel(x_hbm, table_hbm, out_hbm, idx_v, rows_v, sem0, sem1):
        wid = lax.axis_index("s") * _NC + lax.axis_index("c")
        b0 = wid * b_per_w
        sems = (sem0, sem1)

        pltpu.sync_copy(x_hbm.at[pl.ds(b0, b_per_w)], idx_v)

        def fire(r, g):
            for b in range(_K):
                bb = r * _K + b
                pltpu.async_copy(
                    table_hbm.at[idx_v.at[bb]],
                    rows_v.at[g, b],
                    sems[g],
                )

        def drain_and_store(r, g):
            for b in range(_K):
                bb = r * _K + b
                pltpu.make_async_copy(
                    table_hbm.at[idx_v.at[bb]],
                    rows_v.at[g, b],
                    sems[g],
                ).wait()
            for b in range(_K):
                bb = r * _K + b
                pltpu.sync_copy(rows_v.at[g, b], out_hbm.at[b0 + bb])

        fire(0, 0)
        fire(1, 1)

        def body(rr, carry):
            r0 = 2 * rr
            drain_and_store(r0, 0)

            @pl.when(r0 + 2 < n_rounds)
            def _():
                fire(r0 + 2, 0)

            drain_and_store(r0 + 1, 1)

            @pl.when(r0 + 3 < n_rounds)
            def _():
                fire(r0 + 3, 1)

            return carry

        lax.fori_loop(0, n_rounds // 2, body, 0)

    return gather_kernel(x, table)


def kernel(x, W_E):
    table = W_E.T
    return _sc_gather(x, table)


# speedup vs baseline: 1.2201x; 1.2201x over previous
import functools

import jax
import jax.numpy as jnp
from jax import lax
from jax.experimental import pallas as pl
from jax.experimental.pallas import tpu as pltpu
from jax.experimental.pallas import tpu_sc as plsc

_CHUNK = 128
_K = 2
_NC = 2
_NW = 16 * _NC


@jax.jit
def _sc_gather(idx2d, table):
    n_chunks_total, _ = idx2d.shape
    dpad = table.shape[1]
    d = dpad // 2
    n = n_chunks_total * _CHUNK
    c_per_w = n_chunks_total // _NW
    n_rounds = c_per_w // _K
    mesh = plsc.VectorSubcoreMesh(
        core_axis_name="c", subcore_axis_name="s", num_cores=_NC
    )

    @functools.partial(
        pl.kernel,
        out_type=jax.ShapeDtypeStruct((n, dpad), jnp.float32),
        mesh=mesh,
        scratch_types=[
            pltpu.VMEM((c_per_w, _CHUNK), jnp.int32),
            pltpu.VMEM((2, _K, _CHUNK, dpad), jnp.float32),
            pltpu.SemaphoreType.DMA,
            pltpu.SemaphoreType.DMA,
        ],
        compiler_params=pltpu.CompilerParams(
            use_tc_tiling_on_sc=True, skip_device_barrier=True
        ),
    )
    def gather_kernel(idx_hbm, table_hbm, out_hbm, idx_v, rows_v, sem0, sem1):
        wid = lax.axis_index("s") * _NC + lax.axis_index("c")
        chunk0 = wid * c_per_w
        sems = (sem0, sem1)

        pltpu.sync_copy(idx_hbm.at[pl.ds(chunk0, c_per_w)], idx_v)

        def fire(r, g):
            for b in range(_K):
                c = r * _K + b
                pltpu.async_copy(
                    table_hbm.at[idx_v.at[c]],
                    rows_v.at[g, b],
                    sems[g],
                )

        def drain_and_store(r, g):
            for b in range(_K):
                c = r * _K + b
                pltpu.make_async_copy(
                    table_hbm.at[idx_v.at[c]],
                    rows_v.at[g, b],
                    sems[g],
                ).wait()
            for b in range(_K):
                c = r * _K + b
                pltpu.sync_copy(
                    rows_v.at[g, b],
                    out_hbm.at[pl.ds((chunk0 + c) * _CHUNK, _CHUNK)],
                )

        fire(0, 0)
        fire(1, 1)

        def body(rr, carry):
            r0 = 2 * rr
            drain_and_store(r0, 0)

            @pl.when(r0 + 2 < n_rounds)
            def _():
                fire(r0 + 2, 0)

            drain_and_store(r0 + 1, 1)

            @pl.when(r0 + 3 < n_rounds)
            def _():
                fire(r0 + 3, 1)

            return carry

        lax.fori_loop(0, n_rounds // 2, body, 0)

    return gather_kernel(idx2d, table)


def kernel(x, W_E):
    b, p = x.shape
    d, v = W_E.shape
    n = b * p
    table = jnp.pad(W_E.T, ((0, 0), (0, d)))
    idx2d = x.reshape(n // _CHUNK, _CHUNK)
    out = _sc_gather(idx2d, table)
    return out[:, :d].reshape(b, p, d)
